# baseline (speedup 1.0000x reference)
import functools

import jax
import jax.numpy as jnp
from jax import lax
from jax.experimental import pallas as pl
from jax.experimental.pallas import tpu as pltpu
from jax.experimental.pallas import tpu_sc as plsc

N = 50000
D = 128
E = 300000

NPAD = 51200
CHUNK = 5120
CHUNK_PAD = CHUNK + 16
RPT = CHUNK // 16
TPT = 18752
NV = TPT // 16
CAP = TPT + 128

ROWS = 1024


def _rowscat_body(s0_ref, d0_ref, s1_ref, d1_ref, xwn0, xwn1, agg0, agg1,
                  esrc, edst, pbuf, sstage, dstage, rows, acc, sem):
    core = lax.axis_index("c")
    sub = lax.axis_index("s")
    base_u = sub * TPT
    base = jnp.minimum(base_u, E - TPT)
    skip = base_u - base
    lane = lax.broadcasted_iota(jnp.int32, (16,), 0)
    zeros16 = jnp.zeros((16,), jnp.float32)
    dump16 = jnp.full((16,), CHUNK << 16, jnp.int32)

    for s_hbm, d_hbm, xwn, agg in ((s0_ref, d0_ref, xwn0, agg0),
                                   (s1_ref, d1_ref, xwn1, agg1)):
        pltpu.sync_copy(s_hbm.at[pl.ds(base, TPT)], esrc)
        pltpu.sync_copy(d_hbm.at[pl.ds(base, TPT)], edst)
        for k in range(5):
            lo = (core * 5 + k) * CHUNK

            def _zrows(r, _):
                for q in range(8):
                    rows[0, r, pl.ds(q * 16, 16)] = zeros16
                return 0
            lax.fori_loop(0, 128, _zrows, 0)
            for j in range(2):
                pltpu.sync_copy(rows.at[0],
                                acc.at[pl.ds(sub * RPT + j * 128, 128)])
            pltpu.sync_copy(rows.at[0, pl.ds(0, 64)],
                            acc.at[pl.ds(sub * RPT + 256, 64)])
            plsc.subcore_barrier()

            def _comp(i, cnt):
                d = edst[pl.ds(i * 16, 16)]
                s = esrc[pl.ds(i * 16, 16)]
                pos = i * 16 + lane
                m = (d >= lo) & (d < lo + CHUNK) & (pos >= skip)
                pfx = plsc.cumsum(jnp.where(m, 1, 0))
                slot = jnp.where(m, cnt + pfx - 1, CAP - 1)
                plsc.store_scatter(pbuf, [slot], s | ((d - lo) << 16))
                return cnt + jnp.max(pfx)
            cnt = lax.fori_loop(0, NV, _comp, jnp.int32(0))

            for q in range(8):
                pbuf[pl.ds(cnt + q * 16, 16)] = dump16

            nb = (cnt + 127) // 128

            def _issue(j, p):
                for q in range(8):
                    v = pbuf[pl.ds(j * 128 + q * 16, 16)]
                    sstage[p, pl.ds(q * 16, 16)] = v & 0xFFFF
                    dstage[p, pl.ds(q * 16, 16)] = lax.shift_right_logical(v, 16)
                pltpu.async_copy(xwn.at[sstage.at[p]], rows.at[p], sem.at[p])

            @pl.when(nb > 0)
            def _():
                _issue(0, 0)

            def _batch(j, _):
                pc = j & 1
                @pl.when(j + 1 < nb)
                def _():
                    _issue(j + 1, (j + 1) & 1)
                pltpu.make_async_copy(xwn.at[sstage.at[pc]],
                                      rows.at[pc], sem.at[pc]).wait()
                pltpu.sync_copy(rows.at[pc], acc.at[dstage.at[pc]], add=True)
                return 0
            lax.fori_loop(0, nb, _batch, 0)
            plsc.subcore_barrier()

            for j in range(2):
                pltpu.sync_copy(acc.at[pl.ds(sub * RPT + j * 128, 128)],
                                agg.at[pl.ds(lo + sub * RPT + j * 128, 128)])
            pltpu.sync_copy(acc.at[pl.ds(sub * RPT + 256, 64)],
                            agg.at[pl.ds(lo + sub * RPT + 256, 64)])


_rowscat = functools.partial(
    pl.kernel,
    out_type=(jax.ShapeDtypeStruct((NPAD, D), jnp.float32),
              jax.ShapeDtypeStruct((NPAD, D), jnp.float32)),
    mesh=plsc.VectorSubcoreMesh(core_axis_name="c", subcore_axis_name="s"),
    compiler_params=pltpu.CompilerParams(needs_layout_passes=False),
    scratch_types=[
        pltpu.VMEM((TPT,), jnp.int32),
        pltpu.VMEM((TPT,), jnp.int32),
        pltpu.VMEM((CAP,), jnp.int32),
        pltpu.VMEM((2, 128), jnp.int32),
        pltpu.VMEM((2, 128), jnp.int32),
        pltpu.VMEM((2, 128, D), jnp.float32),
        pltpu.VMEM_SHARED((CHUNK_PAD, D), jnp.float32),
        pltpu.SemaphoreType.DMA((2,)),
    ],
)(_rowscat_body)


def _deg_body(s0_ref, d0_ref, s1_ref, d1_ref, out, hist, ebuf, rowidx, sphist):
    core = lax.axis_index("c")
    sub = lax.axis_index("s")
    base_u = sub * TPT
    base = jnp.minimum(base_u, E - TPT)
    skip = base_u - base
    lane = lax.broadcasted_iota(jnp.int32, (16,), 0)
    zeros16 = jnp.zeros((16,), jnp.float32)

    for q in range(4):
        for v in range(8):
            rowidx[q, pl.ds(v * 16, 16)] = q * 128 + v * 16 + lane

    for rel, (a_ref, b_ref) in enumerate(((s0_ref, d0_ref), (s1_ref, d1_ref))):
        for which, idx_hbm in enumerate((a_ref, b_ref)):
            hid = 2 * rel + which

            @pl.when(core == rel)
            def _():
                def _zh(r, _):
                    for q in range(8):
                        hist[r, pl.ds(q * 16, 16)] = zeros16
                    return 0
                lax.fori_loop(0, 512, _zh, 0)
                pltpu.sync_copy(hist.at[pl.ds(sub * 32, 32)],
                                sphist.at[pl.ds(sub * 32, 32)])
                pltpu.sync_copy(idx_hbm.at[pl.ds(base, TPT)], ebuf)

            plsc.subcore_barrier()

            @pl.when(core == rel)
            def _():
                def _acc(i, _):
                    v = ebuf[pl.ds(i * 16, 16)]
                    pos = i * 16 + lane
                    one = jnp.where(pos >= skip, 1.0, 0.0)
                    plsc.addupdate_scatter(hist, [v >> 7, v & 127], one)
                    return 0
                lax.fori_loop(0, NV, _acc, 0)

            plsc.subcore_barrier()

            @pl.when(core == rel)
            def _():
                for q in range(4):
                    pltpu.sync_copy(hist.at[pl.ds(q * 128, 128)],
                                    sphist.at[rowidx.at[q]], add=True)

            plsc.subcore_barrier()

            @pl.when(core == rel)
            def _():
                pltpu.sync_copy(sphist.at[pl.ds(sub * 32, 32)],
                                out.at[hid, pl.ds(sub * 32, 32)])

            plsc.subcore_barrier()


_sc_degrees = functools.partial(
    pl.kernel,
    out_type=jax.ShapeDtypeStruct((4, 512, D), jnp.float32),
    mesh=plsc.VectorSubcoreMesh(core_axis_name="c", subcore_axis_name="s"),
    compiler_params=pltpu.CompilerParams(needs_layout_passes=False),
    scratch_types=[
        pltpu.VMEM((512, D), jnp.float32),
        pltpu.VMEM((TPT,), jnp.int32),
        pltpu.VMEM((4, 128), jnp.int32),
        pltpu.VMEM_SHARED((512, D), jnp.float32),
    ],
)(_deg_body)


WSC = 9376
WPAD = 9472
NBSC = WPAD // 64


def _score_body(s_hbm, d_hbm, h_hbm, out_hbm, sidx, didx, cidx, rsd, obuf, sem):
    core = lax.axis_index("c")
    sub = lax.axis_index("s")
    wid = sub * 2 + core
    base_u = wid * WSC
    base = jnp.minimum(base_u, E - WSC)
    lane = lax.broadcasted_iota(jnp.int32, (16,), 0)
    zeros16i = jnp.zeros((16,), jnp.int32)

    pltpu.sync_copy(s_hbm.at[pl.ds(base, WSC)], sidx.at[pl.ds(0, WSC)])
    pltpu.sync_copy(d_hbm.at[pl.ds(base, WSC)], didx.at[pl.ds(0, WSC)])
    for q in range(6):
        sidx[pl.ds(WSC + q * 16, 16)] = zeros16i
        didx[pl.ds(WSC + q * 16, 16)] = zeros16i

    def _issue(j, p):
        for q in range(4):
            cidx[p, pl.ds(q * 16, 16)] = sidx[pl.ds(j * 64 + q * 16, 16)]
            cidx[p, pl.ds(64 + q * 16, 16)] = didx[pl.ds(j * 64 + q * 16, 16)]
        pltpu.async_copy(h_hbm.at[cidx.at[p]], rsd.at[p], sem.at[p])

    _issue(0, 0)

    def _batch(j, _):
        pc = j & 1
        @pl.when(j + 1 < NBSC)
        def _():
            _issue(j + 1, (j + 1) & 1)
        pltpu.make_async_copy(h_hbm.at[cidx.at[pc]],
                              rsd.at[pc], sem.at[pc]).wait()
        pcv = jnp.full((16,), pc, jnp.int32)

        def _group(g, _):
            e16 = g * 16 + lane
            acc = jnp.zeros((16,), jnp.float32)
            def _col(c, acc):
                for u in range(8):
                    dv = jnp.full((16,), c * 8 + u, jnp.int32)
                    sv = plsc.load_gather(rsd, [pcv, e16, dv])
                    tv = plsc.load_gather(rsd, [pcv, e16 + 64, dv])
                    acc = acc + sv * tv
                return acc
            acc = lax.fori_loop(0, D // 8, _col, acc)
            obuf[pl.ds(j * 64 + g * 16, 16)] = acc
            return 0
        lax.fori_loop(0, 4, _group, 0)
        return 0
    lax.fori_loop(0, NBSC, _batch, 0)
    pltpu.sync_copy(obuf.at[pl.ds(0, WSC)], out_hbm.at[pl.ds(base, WSC)])


_sc_score = functools.partial(
    pl.kernel,
    out_type=jax.ShapeDtypeStruct((E,), jnp.float32),
    mesh=plsc.VectorSubcoreMesh(core_axis_name="c", subcore_axis_name="s"),
    compiler_params=pltpu.CompilerParams(needs_layout_passes=False),
    scratch_types=[
        pltpu.VMEM((WPAD,), jnp.int32),
        pltpu.VMEM((WPAD,), jnp.int32),
        pltpu.VMEM((2, 128), jnp.int32),
        pltpu.VMEM((2, 128, D), jnp.float32),
        pltpu.VMEM((WPAD,), jnp.float32),
        pltpu.SemaphoreType.DMA((2,)),
    ],
)(_score_body)


def _mm1_body(x_ref, w0_ref, w1_ref, cs0_ref, cs1_ref, o0_ref, o1_ref):
    x = x_ref[...]
    o0_ref[...] = (x @ w0_ref[...]) * cs0_ref[...]
    o1_ref[...] = (x @ w1_ref[...]) * cs1_ref[...]


def _tc_mm1(x, W0, W1, cs0, cs1):
    grid = (NPAD // ROWS,)
    return pl.pallas_call(
        _mm1_body,
        grid=grid,
        in_specs=[
            pl.BlockSpec((ROWS, D), lambda i: (i, 0)),
            pl.BlockSpec((D, D), lambda i: (0, 0)),
            pl.BlockSpec((D, D), lambda i: (0, 0)),
            pl.BlockSpec((ROWS, 1), lambda i: (i, 0)),
            pl.BlockSpec((ROWS, 1), lambda i: (i, 0)),
        ],
        out_specs=[
            pl.BlockSpec((ROWS, D), lambda i: (i, 0)),
            pl.BlockSpec((ROWS, D), lambda i: (i, 0)),
        ],
        out_shape=[
            jax.ShapeDtypeStruct((NPAD, D), jnp.float32),
            jax.ShapeDtypeStruct((NPAD, D), jnp.float32),
        ],
    )(x, W0, W1, cs0, cs1)


def _mid_body(a0_ref, a1_ref, cd0_ref, cd1_ref, b0_ref, b1_ref,
              w0_ref, w1_ref, cs0_ref, cs1_ref, o0_ref, o1_ref):
    h = (a0_ref[...] * cd0_ref[...] + b0_ref[...]
         + a1_ref[...] * cd1_ref[...] + b1_ref[...])
    h = jnp.maximum(h, 0.0)
    o0_ref[...] = (h @ w0_ref[...]) * cs0_ref[...]
    o1_ref[...] = (h @ w1_ref[...]) * cs1_ref[...]


def _tc_mid(a0, a1, cd0, cd1, b0, b1, W0, W1, cs0, cs1):
    grid = (NPAD // ROWS,)
    vec = pl.BlockSpec((ROWS, 1), lambda i: (i, 0))
    mat = pl.BlockSpec((ROWS, D), lambda i: (i, 0))
    b = pl.BlockSpec((1, D), lambda i: (0, 0))
    w = pl.BlockSpec((D, D), lambda i: (0, 0))
    return pl.pallas_call(
        _mid_body,
        grid=grid,
        in_specs=[mat, mat, vec, vec, b, b, w, w, vec, vec],
        out_specs=[mat, mat],
        out_shape=[
            jax.ShapeDtypeStruct((NPAD, D), jnp.float32),
            jax.ShapeDtypeStruct((NPAD, D), jnp.float32),
        ],
    )(a0, a1, cd0, cd1, b0, b1, W0, W1, cs0, cs1)


def _fin_body(a0_ref, a1_ref, cd0_ref, cd1_ref, b0_ref, b1_ref, o_ref):
    o_ref[...] = (a0_ref[...] * cd0_ref[...] + b0_ref[...]
                  + a1_ref[...] * cd1_ref[...] + b1_ref[...])


def _tc_fin(a0, a1, cd0, cd1, b0, b1):
    grid = (NPAD // ROWS,)
    vec = pl.BlockSpec((ROWS, 1), lambda i: (i, 0))
    mat = pl.BlockSpec((ROWS, D), lambda i: (i, 0))
    b = pl.BlockSpec((1, D), lambda i: (0, 0))
    return pl.pallas_call(
        _fin_body,
        grid=grid,
        in_specs=[mat, mat, vec, vec, b, b],
        out_specs=mat,
        out_shape=jax.ShapeDtypeStruct((NPAD, D), jnp.float32),
    )(a0, a1, cd0, cd1, b0, b1)


def _norm(deg):
    return jnp.where(deg > 0, lax.rsqrt(jnp.maximum(deg, 1.0)), 0.0)


def kernel(x, edge_index_r0, edge_index_r1, W1_r0, b1_r0, W1_r1, b1_r1,
           W2_r0, b2_r0, W2_r1, b2_r1):
    s0, d0 = edge_index_r0[0], edge_index_r0[1]
    s1, d1 = edge_index_r1[0], edge_index_r1[1]

    deg = _sc_degrees(s0, d0, s1, d1).reshape(4, 512 * D)
    deg_s0, deg_d0, deg_s1, deg_d1 = (deg[i, :N] for i in range(4))

    pad = NPAD - N
    cs0 = jnp.pad(_norm(deg_s0), (0, pad))[:, None]
    cd0 = jnp.pad(_norm(deg_d0), (0, pad))[:, None]
    cs1 = jnp.pad(_norm(deg_s1), (0, pad))[:, None]
    cd1 = jnp.pad(_norm(deg_d1), (0, pad))[:, None]
    xp = jnp.pad(x, ((0, pad), (0, 0)))

    b1_r0 = b1_r0[None, :]
    b1_r1 = b1_r1[None, :]
    b2_r0 = b2_r0[None, :]
    b2_r1 = b2_r1[None, :]

    xwn0, xwn1 = _tc_mm1(xp, W1_r0, W1_r1, cs0, cs1)
    agg0, agg1 = _rowscat(s0, d0, s1, d1, xwn0, xwn1)

    xw20, xw21 = _tc_mid(agg0, agg1, cd0, cd1, b1_r0, b1_r1,
                         W2_r0, W2_r1, cs0, cs1)
    agg20, agg21 = _rowscat(s0, d0, s1, d1, xw20, xw21)

    h2 = _tc_fin(agg20, agg21, cd0, cd1, b2_r0, b2_r1)

    score = _sc_score(s0, d0, h2)[:, None]
    return score


# speedup vs baseline: 1.0182x; 1.0182x over previous
import functools

import jax
import jax.numpy as jnp
from jax import lax
from jax.experimental import pallas as pl
from jax.experimental.pallas import tpu as pltpu
from jax.experimental.pallas import tpu_sc as plsc

N = 50000
D = 128
E = 300000

NPAD = 51200
CHUNK = 6400
CHUNK_PAD = CHUNK + 16
RPT = CHUNK // 16
TPT = 18752
NV = TPT // 16
CAP = TPT + 128

ROWS = 1024


def _rowscat_body(s0_ref, d0_ref, s1_ref, d1_ref, xwn0, xwn1, agg0, agg1,
                  esrc, edst, pbuf, sstage, dstage, rows, acc, sem):
    core = lax.axis_index("c")
    sub = lax.axis_index("s")
    base_u = sub * TPT
    base = jnp.minimum(base_u, E - TPT)
    skip = base_u - base
    lane = lax.broadcasted_iota(jnp.int32, (16,), 0)
    zeros16 = jnp.zeros((16,), jnp.float32)
    dump16 = jnp.full((16,), CHUNK << 16, jnp.int32)

    for s_hbm, d_hbm, xwn, agg in ((s0_ref, d0_ref, xwn0, agg0),
                                   (s1_ref, d1_ref, xwn1, agg1)):
        pltpu.sync_copy(s_hbm.at[pl.ds(base, TPT)], esrc)
        pltpu.sync_copy(d_hbm.at[pl.ds(base, TPT)], edst)
        for k in range(4):
            lo = (core * 4 + k) * CHUNK

            def _zrows(r, _):
                for q in range(8):
                    rows[r, pl.ds(q * 16, 16)] = zeros16
                return 0
            lax.fori_loop(0, 128, _zrows, 0)
            for j in range(3):
                pltpu.sync_copy(rows, acc.at[pl.ds(sub * RPT + j * 128, 128)])
            pltpu.sync_copy(rows.at[pl.ds(0, 16)],
                            acc.at[pl.ds(sub * RPT + 384, 16)])
            plsc.subcore_barrier()

            def _comp(i, cnt):
                d = edst[pl.ds(i * 16, 16)]
                s = esrc[pl.ds(i * 16, 16)]
                pos = i * 16 + lane
                m = (d >= lo) & (d < lo + CHUNK) & (pos >= skip)
                pfx = plsc.cumsum(jnp.where(m, 1, 0))
                slot = jnp.where(m, cnt + pfx - 1, CAP - 1)
                plsc.store_scatter(pbuf, [slot], s | ((d - lo) << 16))
                return cnt + jnp.max(pfx)
            cnt = lax.fori_loop(0, NV, _comp, jnp.int32(0))

            for q in range(8):
                pbuf[pl.ds(cnt + q * 16, 16)] = dump16

            def _batch(j, _):
                for q in range(8):
                    v = pbuf[pl.ds(j * 128 + q * 16, 16)]
                    sstage[pl.ds(q * 16, 16)] = v & 0xFFFF
                    dstage[pl.ds(q * 16, 16)] = lax.shift_right_logical(v, 16)
                pltpu.async_copy(xwn.at[sstage], rows, sem).wait()
                pltpu.sync_copy(rows, acc.at[dstage], add=True)
                return 0
            lax.fori_loop(0, (cnt + 127) // 128, _batch, 0)
            plsc.subcore_barrier()

            for j in range(3):
                pltpu.sync_copy(acc.at[pl.ds(sub * RPT + j * 128, 128)],
                                agg.at[pl.ds(lo + sub * RPT + j * 128, 128)])
            pltpu.sync_copy(acc.at[pl.ds(sub * RPT + 384, 16)],
                            agg.at[pl.ds(lo + sub * RPT + 384, 16)])


_rowscat = functools.partial(
    pl.kernel,
    out_type=(jax.ShapeDtypeStruct((NPAD, D), jnp.float32),
              jax.ShapeDtypeStruct((NPAD, D), jnp.float32)),
    mesh=plsc.VectorSubcoreMesh(core_axis_name="c", subcore_axis_name="s"),
    compiler_params=pltpu.CompilerParams(needs_layout_passes=False),
    scratch_types=[
        pltpu.VMEM((TPT,), jnp.int32),
        pltpu.VMEM((TPT,), jnp.int32),
        pltpu.VMEM((CAP,), jnp.int32),
        pltpu.VMEM((128,), jnp.int32),
        pltpu.VMEM((128,), jnp.int32),
        pltpu.VMEM((128, D), jnp.float32),
        pltpu.VMEM_SHARED((CHUNK_PAD, D), jnp.float32),
        pltpu.SemaphoreType.DMA,
    ],
)(_rowscat_body)


def _deg_body(s0_ref, d0_ref, s1_ref, d1_ref, out, hist, ebuf, rowidx, sphist):
    core = lax.axis_index("c")
    sub = lax.axis_index("s")
    base_u = sub * TPT
    base = jnp.minimum(base_u, E - TPT)
    skip = base_u - base
    lane = lax.broadcasted_iota(jnp.int32, (16,), 0)
    zeros16 = jnp.zeros((16,), jnp.float32)

    for q in range(4):
        for v in range(8):
            rowidx[q, pl.ds(v * 16, 16)] = q * 128 + v * 16 + lane

    for rel, (a_ref, b_ref) in enumerate(((s0_ref, d0_ref), (s1_ref, d1_ref))):
        for which, idx_hbm in enumerate((a_ref, b_ref)):
            hid = 2 * rel + which

            @pl.when(core == rel)
            def _():
                def _zh(r, _):
                    for q in range(8):
                        hist[r, pl.ds(q * 16, 16)] = zeros16
                    return 0
                lax.fori_loop(0, 512, _zh, 0)
                pltpu.sync_copy(hist.at[pl.ds(sub * 32, 32)],
                                sphist.at[pl.ds(sub * 32, 32)])
                pltpu.sync_copy(idx_hbm.at[pl.ds(base, TPT)], ebuf)

            plsc.subcore_barrier()

            @pl.when(core == rel)
            def _():
                def _acc(i, _):
                    v = ebuf[pl.ds(i * 16, 16)]
                    pos = i * 16 + lane
                    one = jnp.where(pos >= skip, 1.0, 0.0)
                    plsc.addupdate_scatter(hist, [v >> 7, v & 127], one)
                    return 0
                lax.fori_loop(0, NV, _acc, 0)

            plsc.subcore_barrier()

            @pl.when(core == rel)
            def _():
                for q in range(4):
                    pltpu.sync_copy(hist.at[pl.ds(q * 128, 128)],
                                    sphist.at[rowidx.at[q]], add=True)

            plsc.subcore_barrier()

            @pl.when(core == rel)
            def _():
                pltpu.sync_copy(sphist.at[pl.ds(sub * 32, 32)],
                                out.at[hid, pl.ds(sub * 32, 32)])

            plsc.subcore_barrier()


_sc_degrees = functools.partial(
    pl.kernel,
    out_type=jax.ShapeDtypeStruct((4, 512, D), jnp.float32),
    mesh=plsc.VectorSubcoreMesh(core_axis_name="c", subcore_axis_name="s"),
    compiler_params=pltpu.CompilerParams(needs_layout_passes=False),
    scratch_types=[
        pltpu.VMEM((512, D), jnp.float32),
        pltpu.VMEM((TPT,), jnp.int32),
        pltpu.VMEM((4, 128), jnp.int32),
        pltpu.VMEM_SHARED((512, D), jnp.float32),
    ],
)(_deg_body)


WSC = 9376
WPAD = 9472
NBSC = WPAD // 64


def _score_body(s_hbm, d_hbm, h_hbm, out_hbm, sidx, didx, cidx, rsd, obuf, sem):
    core = lax.axis_index("c")
    sub = lax.axis_index("s")
    wid = sub * 2 + core
    base_u = wid * WSC
    base = jnp.minimum(base_u, E - WSC)
    lane = lax.broadcasted_iota(jnp.int32, (16,), 0)
    zeros16i = jnp.zeros((16,), jnp.int32)

    pltpu.sync_copy(s_hbm.at[pl.ds(base, WSC)], sidx.at[pl.ds(0, WSC)])
    pltpu.sync_copy(d_hbm.at[pl.ds(base, WSC)], didx.at[pl.ds(0, WSC)])
    for q in range(6):
        sidx[pl.ds(WSC + q * 16, 16)] = zeros16i
        didx[pl.ds(WSC + q * 16, 16)] = zeros16i

    def _issue(j, p):
        for q in range(4):
            cidx[p, pl.ds(q * 16, 16)] = sidx[pl.ds(j * 64 + q * 16, 16)]
            cidx[p, pl.ds(64 + q * 16, 16)] = didx[pl.ds(j * 64 + q * 16, 16)]
        pltpu.async_copy(h_hbm.at[cidx.at[p]], rsd.at[p], sem.at[p])

    _issue(0, 0)
    _issue(1, 1)

    def _batch(j, _):
        pc = j % 3
        @pl.when(j + 2 < NBSC)
        def _():
            _issue(j + 2, (j + 2) % 3)
        pltpu.make_async_copy(h_hbm.at[cidx.at[pc]],
                              rsd.at[pc], sem.at[pc]).wait()
        pcv = jnp.full((16,), pc, jnp.int32)

        def _group(g, _):
            e16 = g * 16 + lane
            acc = jnp.zeros((16,), jnp.float32)
            def _col(c, acc):
                for u in range(8):
                    dv = jnp.full((16,), c * 8 + u, jnp.int32)
                    sv = plsc.load_gather(rsd, [pcv, e16, dv])
                    tv = plsc.load_gather(rsd, [pcv, e16 + 64, dv])
                    acc = acc + sv * tv
                return acc
            acc = lax.fori_loop(0, D // 8, _col, acc)
            obuf[pl.ds(j * 64 + g * 16, 16)] = acc
            return 0
        lax.fori_loop(0, 4, _group, 0)
        return 0
    lax.fori_loop(0, NBSC, _batch, 0)
    pltpu.sync_copy(obuf.at[pl.ds(0, WSC)], out_hbm.at[pl.ds(base, WSC)])


_sc_score = functools.partial(
    pl.kernel,
    out_type=jax.ShapeDtypeStruct((E,), jnp.float32),
    mesh=plsc.VectorSubcoreMesh(core_axis_name="c", subcore_axis_name="s"),
    compiler_params=pltpu.CompilerParams(needs_layout_passes=False),
    scratch_types=[
        pltpu.VMEM((WPAD,), jnp.int32),
        pltpu.VMEM((WPAD,), jnp.int32),
        pltpu.VMEM((3, 128), jnp.int32),
        pltpu.VMEM((3, 128, D), jnp.float32),
        pltpu.VMEM((WPAD,), jnp.float32),
        pltpu.SemaphoreType.DMA((3,)),
    ],
)(_score_body)


def _mm1_body(x_ref, w0_ref, w1_ref, cs0_ref, cs1_ref, o0_ref, o1_ref):
    x = x_ref[...]
    o0_ref[...] = (x @ w0_ref[...]) * cs0_ref[...]
    o1_ref[...] = (x @ w1_ref[...]) * cs1_ref[...]


def _tc_mm1(x, W0, W1, cs0, cs1):
    grid = (NPAD // ROWS,)
    return pl.pallas_call(
        _mm1_body,
        grid=grid,
        in_specs=[
            pl.BlockSpec((ROWS, D), lambda i: (i, 0)),
            pl.BlockSpec((D, D), lambda i: (0, 0)),
            pl.BlockSpec((D, D), lambda i: (0, 0)),
            pl.BlockSpec((ROWS, 1), lambda i: (i, 0)),
            pl.BlockSpec((ROWS, 1), lambda i: (i, 0)),
        ],
        out_specs=[
            pl.BlockSpec((ROWS, D), lambda i: (i, 0)),
            pl.BlockSpec((ROWS, D), lambda i: (i, 0)),
        ],
        out_shape=[
            jax.ShapeDtypeStruct((NPAD, D), jnp.float32),
            jax.ShapeDtypeStruct((NPAD, D), jnp.float32),
        ],
    )(x, W0, W1, cs0, cs1)


def _mid_body(a0_ref, a1_ref, cd0_ref, cd1_ref, b0_ref, b1_ref,
              w0_ref, w1_ref, cs0_ref, cs1_ref, o0_ref, o1_ref):
    h = (a0_ref[...] * cd0_ref[...] + b0_ref[...]
         + a1_ref[...] * cd1_ref[...] + b1_ref[...])
    h = jnp.maximum(h, 0.0)
    o0_ref[...] = (h @ w0_ref[...]) * cs0_ref[...]
    o1_ref[...] = (h @ w1_ref[...]) * cs1_ref[...]


def _tc_mid(a0, a1, cd0, cd1, b0, b1, W0, W1, cs0, cs1):
    grid = (NPAD // ROWS,)
    vec = pl.BlockSpec((ROWS, 1), lambda i: (i, 0))
    mat = pl.BlockSpec((ROWS, D), lambda i: (i, 0))
    b = pl.BlockSpec((1, D), lambda i: (0, 0))
    w = pl.BlockSpec((D, D), lambda i: (0, 0))
    return pl.pallas_call(
        _mid_body,
        grid=grid,
        in_specs=[mat, mat, vec, vec, b, b, w, w, vec, vec],
        out_specs=[mat, mat],
        out_shape=[
            jax.ShapeDtypeStruct((NPAD, D), jnp.float32),
            jax.ShapeDtypeStruct((NPAD, D), jnp.float32),
        ],
    )(a0, a1, cd0, cd1, b0, b1, W0, W1, cs0, cs1)


def _fin_body(a0_ref, a1_ref, cd0_ref, cd1_ref, b0_ref, b1_ref, o_ref):
    o_ref[...] = (a0_ref[...] * cd0_ref[...] + b0_ref[...]
                  + a1_ref[...] * cd1_ref[...] + b1_ref[...])


def _tc_fin(a0, a1, cd0, cd1, b0, b1):
    grid = (NPAD // ROWS,)
    vec = pl.BlockSpec((ROWS, 1), lambda i: (i, 0))
    mat = pl.BlockSpec((ROWS, D), lambda i: (i, 0))
    b = pl.BlockSpec((1, D), lambda i: (0, 0))
    return pl.pallas_call(
        _fin_body,
        grid=grid,
        in_specs=[mat, mat, vec, vec, b, b],
        out_specs=mat,
        out_shape=jax.ShapeDtypeStruct((NPAD, D), jnp.float32),
    )(a0, a1, cd0, cd1, b0, b1)


def _norm(deg):
    return jnp.where(deg > 0, lax.rsqrt(jnp.maximum(deg, 1.0)), 0.0)


def kernel(x, edge_index_r0, edge_index_r1, W1_r0, b1_r0, W1_r1, b1_r1,
           W2_r0, b2_r0, W2_r1, b2_r1):
    s0, d0 = edge_index_r0[0], edge_index_r0[1]
    s1, d1 = edge_index_r1[0], edge_index_r1[1]

    deg = _sc_degrees(s0, d0, s1, d1).reshape(4, 512 * D)
    deg_s0, deg_d0, deg_s1, deg_d1 = (deg[i, :N] for i in range(4))

    pad = NPAD - N
    cs0 = jnp.pad(_norm(deg_s0), (0, pad))[:, None]
    cd0 = jnp.pad(_norm(deg_d0), (0, pad))[:, None]
    cs1 = jnp.pad(_norm(deg_s1), (0, pad))[:, None]
    cd1 = jnp.pad(_norm(deg_d1), (0, pad))[:, None]
    xp = jnp.pad(x, ((0, pad), (0, 0)))

    b1_r0 = b1_r0[None, :]
    b1_r1 = b1_r1[None, :]
    b2_r0 = b2_r0[None, :]
    b2_r1 = b2_r1[None, :]

    xwn0, xwn1 = _tc_mm1(xp, W1_r0, W1_r1, cs0, cs1)
    agg0, agg1 = _rowscat(s0, d0, s1, d1, xwn0, xwn1)

    xw20, xw21 = _tc_mid(agg0, agg1, cd0, cd1, b1_r0, b1_r1,
                         W2_r0, W2_r1, cs0, cs1)
    agg20, agg21 = _rowscat(s0, d0, s1, d1, xw20, xw21)

    h2 = _tc_fin(agg20, agg21, cd0, cd1, b2_r0, b2_r1)

    score = _sc_score(s0, d0, h2)[:, None]
    return score
